# Initial kernel scaffold; baseline (speedup 1.0000x reference)
import jax
import jax.numpy as jnp
from jax.experimental import pallas as pl


def kernel(x1, x2, edge_index1, edge_index2, params):
    raise NotImplementedError("write your pallas kernel here")


# speedup vs baseline: 2.9622x; 2.9622x over previous
import functools

import jax
import jax.numpy as jnp
from jax import lax
from jax.experimental import pallas as pl
from jax.experimental.pallas import tpu as pltpu
from jax.experimental.pallas import tpu_sc as plsc

NV = 10000
NE = 320000
DIN = 128
H1 = 128
H2 = 256
KTOP = 1024

NC, NS, LN = 2, 16, 16
NPAD = 10240
DAUG = 144
CHUNK = 256
EPAD = 323584
NCHUNK = EPAD // NS // CHUNK
RPS = NPAD // NS
ZR = 64
BR = 128
COMP_OUT = KTOP + NC * NS * LN

_f32 = jnp.float32
_i32 = jnp.int32


def _sc_mesh():
    return plsc.VectorSubcoreMesh(
        core_axis_name="c", subcore_axis_name="s", num_cores=NC, num_subcores=NS)


def _make_segsum(with_cnt):
    d = 128
    out_type = [jax.ShapeDtypeStruct((NC, NPAD, d), _f32)]
    if with_cnt:
        out_type.append(jax.ShapeDtypeStruct((NC, NPAD), _f32))
    scratch = [
        pltpu.VMEM((CHUNK,), _i32),
        pltpu.VMEM((CHUNK,), _i32),
        pltpu.VMEM((CHUNK, d), _f32),
        pltpu.VMEM((ZR, d), _f32),
        pltpu.VMEM_SHARED((NPAD, d), _f32),
    ]
    if with_cnt:
        scratch += [
            pltpu.VMEM((CHUNK,), _f32),
            pltpu.VMEM((RPS,), _f32),
            pltpu.VMEM_SHARED((NPAD,), _f32),
        ]
    scratch.append(pltpu.SemaphoreType.DMA)

    @functools.partial(pl.kernel, out_type=tuple(out_type), mesh=_sc_mesh(),
                       scratch_types=scratch)
    def k(tab0, tab1, src0, dst0, src1, dst1, *rest):
        if with_cnt:
            (agg_out, cnt_out, sidx, didx, rows, zbuf, acc,
             ones, cstage, cacc, sem) = rest
        else:
            agg_out, sidx, didx, rows, zbuf, acc, sem = rest
        c = lax.axis_index("c")
        s = lax.axis_index("s")

        nz = (ZR * d) // LN
        def zstep(i, _):
            r = i // (d // LN)
            col = (i % (d // LN)) * LN
            zbuf[r, pl.ds(col, LN)] = jnp.zeros((LN,), _f32)
            return 0
        lax.fori_loop(0, nz, zstep, 0)
        base_r = s * RPS
        def zcp(j, _):
            pltpu.sync_copy(zbuf, acc.at[pl.ds(base_r + j * ZR, ZR)])
            return 0
        lax.fori_loop(0, RPS // ZR, zcp, 0)
        if with_cnt:
            def ofill(i, _):
                ones[pl.ds(i * LN, LN)] = jnp.ones((LN,), _f32)
                return 0
            lax.fori_loop(0, CHUNK // LN, ofill, 0)
            def cfill(i, _):
                cstage[pl.ds(i * LN, LN)] = jnp.zeros((LN,), _f32)
                return 0
            lax.fori_loop(0, RPS // LN, cfill, 0)
            pltpu.sync_copy(cstage, cacc.at[pl.ds(base_r, RPS)])
        plsc.subcore_barrier()

        ebase = s * (EPAD // NS)
        def do_edges(tab, srcr, dstr):
            def step(kk, _):
                off = ebase + kk * CHUNK
                pltpu.sync_copy(srcr.at[pl.ds(off, CHUNK)], sidx)
                pltpu.sync_copy(dstr.at[pl.ds(off, CHUNK)], didx)
                pltpu.async_copy(tab.at[sidx], rows, sem).wait()
                pltpu.sync_copy(rows, acc.at[didx], add=True)
                if with_cnt:
                    pltpu.sync_copy(ones, cacc.at[didx], add=True)
                return 0
            lax.fori_loop(0, NCHUNK, step, 0)

        @pl.when(c == 0)
        def _():
            do_edges(tab0, src0, dst0)

        @pl.when(c == 1)
        def _():
            do_edges(tab1, src1, dst1)

        plsc.subcore_barrier()

        pltpu.sync_copy(acc.at[pl.ds(base_r, RPS)],
                        agg_out.at[c, pl.ds(base_r, RPS)])
        if with_cnt:
            pltpu.sync_copy(cacc.at[pl.ds(base_r, RPS)],
                            cnt_out.at[c, pl.ds(base_r, RPS)])

    return k


def _layer1_body(agg, cnt, xa1, xa2, wl, wr, b, h1a, h1b):
    c0 = cnt[0]
    c1 = cnt[1]
    inv0 = 1.0 / jnp.maximum(c0, 1.0)
    inv1 = 1.0 / jnp.maximum(c1, 1.0)
    wlv = wl[...]
    wrv = wr[...]
    bv = b[...]
    h1a[...] = jnp.maximum(
        jnp.dot(agg[0] * inv0, wlv, preferred_element_type=_f32)
        + jnp.dot(xa1[...], wrv, preferred_element_type=_f32) + bv, 0.0)
    h1b[...] = jnp.maximum(
        jnp.dot(agg[1] * inv1, wlv, preferred_element_type=_f32)
        + jnp.dot(xa2[...], wrv, preferred_element_type=_f32) + bv, 0.0)


def _layer2_body(agg, cnt, h1a, h1b, wl, wr, b, sim):
    i = pl.program_id(0)
    c0 = cnt[0]
    c1 = cnt[1]
    inv0 = 1.0 / jnp.maximum(c0, 1.0)
    inv1 = 1.0 / jnp.maximum(c1, 1.0)
    wlv = wl[...]
    wrv = wr[...]
    bv = b[...]
    o1 = jnp.maximum(
        jnp.dot(agg[0] * inv0, wlv, preferred_element_type=_f32)
        + jnp.dot(h1a[...], wrv, preferred_element_type=_f32) + bv, 0.0)
    o2 = jnp.maximum(
        jnp.dot(agg[1] * inv1, wlv, preferred_element_type=_f32)
        + jnp.dot(h1b[...], wrv, preferred_element_type=_f32) + bv, 0.0)
    d = o1 - o2 + 1e-6
    s2 = jnp.sum(d * d, axis=1, keepdims=True)
    row = i * BR + lax.broadcasted_iota(_i32, (BR, 1), 0)
    sim[...] = jnp.where(row < NV, jnp.sqrt(s2), -1.0)


def _bisect_body(simr, t_out, off_out):
    v = simr[...]
    bu = lax.bitcast_convert_type(v, jnp.uint32)
    neg = (bu >> jnp.uint32(31)) == jnp.uint32(1)
    ku = bu ^ jnp.where(neg, jnp.uint32(0xFFFFFFFF), jnp.uint32(0x80000000))

    def step(i, t):
        cand = t | (jnp.uint32(1) << (jnp.uint32(31) - i.astype(jnp.uint32)))
        cnt = jnp.sum((ku >= cand).astype(_i32))
        return jnp.where(cnt >= KTOP, cand, t)

    t = lax.fori_loop(0, 32, step, jnp.uint32(0))
    tb = jnp.where((t >> jnp.uint32(31)) == jnp.uint32(1),
                   t ^ jnp.uint32(0x80000000), t ^ jnp.uint32(0xFFFFFFFF))
    tf = lax.bitcast_convert_type(tb, _f32)
    t_out[...] = jnp.broadcast_to(tf, (1, 1))

    c_sl = jnp.sum(jnp.where(v > tf, 1.0, 0.0), axis=1)
    ri = lax.broadcasted_iota(_i32, (NS, NS), 0)
    tril = jnp.where(ri > lax.broadcasted_iota(_i32, (NS, NS), 1), 1.0, 0.0)
    colsum = jnp.sum(c_sl, axis=0, keepdims=True)
    acol = jnp.dot(colsum, tril.T, preferred_element_type=_f32)
    brow = jnp.dot(tril, c_sl, preferred_element_type=_f32)
    off_out[...] = (acol + brow).astype(_i32)


def _make_compact():
    vps = NPAD // NS
    nvr = vps // LN
    fill = COMP_OUT // NS

    scratch = [
        pltpu.VMEM((vps,), _f32),
        pltpu.VMEM((LN,), _f32),
        pltpu.VMEM((vps,), _f32),
        pltpu.VMEM((vps,), _i32),
        pltpu.VMEM((fill,), _f32),
        pltpu.VMEM((LN,), _i32),
        pltpu.SemaphoreType.DMA,
    ]

    @functools.partial(
        pl.kernel,
        out_type=jax.ShapeDtypeStruct((COMP_OUT,), _f32),
        mesh=_sc_mesh(), scratch_types=scratch)
    def k(sim_hbm, t_hbm, off_hbm, cand_out, vbuf, tbuf, svals, sidx, fbuf,
          offb, sem):
        c = lax.axis_index("c")
        s = lax.axis_index("s")
        zeros = jnp.zeros((LN,), _i32)
        ones = jnp.ones((LN,), _i32)

        @pl.when(c == 0)
        def _():
            pltpu.sync_copy(t_hbm, tbuf)
            tvec = tbuf[...]
            def fstep(i, _):
                fbuf[pl.ds(i * LN, LN)] = tvec
                return 0
            lax.fori_loop(0, fill // LN, fstep, 0)
            pltpu.sync_copy(fbuf, cand_out.at[pl.ds(s * fill, fill)])
            pltpu.sync_copy(sim_hbm.at[pl.ds(s * vps, vps)], vbuf)
            pltpu.sync_copy(off_hbm.at[s], offb)
            plsc.subcore_barrier()

            dump = KTOP + s * LN + lax.iota(_i32, LN)
            def sstep(i, off):
                v = vbuf[pl.ds(i * LN, LN)]
                m = v > tvec
                idx = jnp.where(m, off, dump)
                svals[pl.ds(i * LN, LN)] = v
                sidx[pl.ds(i * LN, LN)] = idx
                return off + jnp.where(m, ones, zeros)
            lax.fori_loop(0, nvr, sstep, offb[...])
            pltpu.sync_copy(svals, cand_out.at[sidx])

    return k


def _rank_body(cv_ref, cs_ref, r_ref):
    cv = cv_ref[...]
    ids = (lax.broadcasted_iota(_i32, (8, 128), 0) * 128
           + lax.broadcasted_iota(_i32, (8, 128), 1))

    def step(j, acc):
        tj = cs_ref[j]
        gt = (tj > cv).astype(_i32)
        eq = (jnp.logical_and(tj == cv, ids > j)).astype(_i32)
        return acc + gt + eq

    r_ref[...] = lax.fori_loop(0, KTOP, step, jnp.zeros((8, 128), _i32))


def _head_body(cs_ref, rs_ref, w1, b1, w2, b2, w3, b3, out_ref):
    pids = (lax.broadcasted_iota(_i32, (8, 128), 0) * 128
            + lax.broadcasted_iota(_i32, (8, 128), 1))

    def step(i, sv):
        return jnp.where(pids == rs_ref[i], cs_ref[i], sv)

    sorted_v = lax.fori_loop(0, KTOP, step, jnp.zeros((8, 128), _f32))
    h = jnp.zeros((1, 128), _f32)
    for si in range(8):
        h = h + jnp.dot(sorted_v[si:si + 1, :], w1[si],
                        preferred_element_type=_f32)
    h = jnp.maximum(h + b1[...], 0.0)
    h = jnp.maximum(jnp.dot(h, w2[...], preferred_element_type=_f32) + b2[...], 0.0)
    o = jnp.dot(h, w3[...], preferred_element_type=_f32) + b3[...]
    out_ref[...] = 1.0 / (1.0 + jnp.exp(-o))


_segsum_l1 = _make_segsum(True)
_segsum_l2 = _make_segsum(False)
_compact = _make_compact()

_layer1 = pl.pallas_call(
    _layer1_body,
    grid=(NPAD // BR,),
    in_specs=[
        pl.BlockSpec((NC, BR, 128), lambda i: (0, i, 0)),
        pl.BlockSpec((NC, BR, 1), lambda i: (0, i, 0)),
        pl.BlockSpec((BR, 128), lambda i: (i, 0)),
        pl.BlockSpec((BR, 128), lambda i: (i, 0)),
        pl.BlockSpec((128, 128), lambda i: (0, 0)),
        pl.BlockSpec((128, 128), lambda i: (0, 0)),
        pl.BlockSpec((1, 128), lambda i: (0, 0)),
    ],
    out_specs=[
        pl.BlockSpec((BR, 128), lambda i: (i, 0)),
        pl.BlockSpec((BR, 128), lambda i: (i, 0)),
    ],
    out_shape=[
        jax.ShapeDtypeStruct((NPAD, 128), _f32),
        jax.ShapeDtypeStruct((NPAD, 128), _f32),
    ],
)

_layer2 = pl.pallas_call(
    _layer2_body,
    grid=(NPAD // BR,),
    in_specs=[
        pl.BlockSpec((NC, BR, 128), lambda i: (0, i, 0)),
        pl.BlockSpec((NC, BR, 1), lambda i: (0, i, 0)),
        pl.BlockSpec((BR, 128), lambda i: (i, 0)),
        pl.BlockSpec((BR, 128), lambda i: (i, 0)),
        pl.BlockSpec((128, H2), lambda i: (0, 0)),
        pl.BlockSpec((128, H2), lambda i: (0, 0)),
        pl.BlockSpec((1, H2), lambda i: (0, 0)),
    ],
    out_specs=pl.BlockSpec((BR, 1), lambda i: (i, 0)),
    out_shape=jax.ShapeDtypeStruct((NPAD, 1), _f32),
)

_bisect = pl.pallas_call(
    _bisect_body,
    out_shape=[
        jax.ShapeDtypeStruct((1, 1), _f32),
        jax.ShapeDtypeStruct((NS, LN), _i32),
    ],
)

_rank = pl.pallas_call(
    _rank_body,
    in_specs=[
        pl.BlockSpec(memory_space=pltpu.VMEM),
        pl.BlockSpec(memory_space=pltpu.SMEM),
    ],
    out_shape=jax.ShapeDtypeStruct((8, 128), _i32),
)

_head = pl.pallas_call(
    _head_body,
    in_specs=[
        pl.BlockSpec(memory_space=pltpu.SMEM),
        pl.BlockSpec(memory_space=pltpu.SMEM),
    ] + [pl.BlockSpec(memory_space=pltpu.VMEM)] * 6,
    out_shape=jax.ShapeDtypeStruct((1, 1), _f32),
)


def _pad_edges(ei):
    npe = EPAD - NE
    ar = jnp.arange(npe, dtype=_i32)
    src = jnp.concatenate([ei[0], ar % NV])
    dst = jnp.concatenate([ei[1], NV + (ar % (NPAD - NV))])
    return src, dst


def kernel(x1, x2, edge_index1, edge_index2, params):
    p = params
    x1p = jnp.pad(x1, ((0, NPAD - NV), (0, 0)))
    x2p = jnp.pad(x2, ((0, NPAD - NV), (0, 0)))
    s1, d1 = _pad_edges(edge_index1)
    s2, d2 = _pad_edges(edge_index2)

    wl1 = p['Wl1'].T
    wr1 = p['Wr1'].T
    b1 = p['b1'].reshape(1, 128)
    wl2 = p['Wl2'].T
    wr2 = p['Wr2'].T
    b2 = p['b2'].reshape(1, H2)
    w1eff = (p['W_fc1'].T * p['g1'][None, :]).reshape(8, 128, 128)
    b1eff = (p['b_fc1'] * p['g1'] + p['be1']).reshape(1, 128)
    w2eff = p['W_fc2'].T * p['g2'][None, :]
    b2eff = (p['b_fc2'] * p['g2'] + p['be2']).reshape(1, H2)
    w3 = p['W_fc3'].T
    b3 = p['b_fc3'].reshape(1, 1)

    agg1, cnt2 = _segsum_l1(x1, x2, s1, d1, s2, d2)
    cnt = cnt2.reshape(NC, NPAD, 1)
    h1a, h1b = _layer1(agg1, cnt, x1p, x2p, wl1, wr1, b1)
    agg2, = _segsum_l2(h1a, h1b, s1, d1, s2, d2)
    sim = _layer2(agg2, cnt, h1a, h1b, wl2, wr2, b2)

    t, off0 = _bisect(sim.reshape(NS, NPAD // NS // LN, LN))
    cand = _compact(sim.reshape(NPAD), jnp.broadcast_to(t.reshape(()), (LN,)),
                    off0)
    c1024 = cand[:KTOP]
    r = _rank(c1024.reshape(8, 128), c1024)
    out = _head(c1024, r.reshape(KTOP), w1eff, b1eff, w2eff, b2eff, w3, b3)
    return out.reshape(1)


# speedup vs baseline: 5.2262x; 1.7643x over previous
import functools

import jax
import jax.numpy as jnp
from jax import lax
from jax.experimental import pallas as pl
from jax.experimental.pallas import tpu as pltpu
from jax.experimental.pallas import tpu_sc as plsc

NV = 10000
NE = 320000
DIN = 128
H1 = 128
H2 = 256
KTOP = 1024

NC, NS, LN = 2, 16, 16
NPAD = 10240
DAUG = 144
CHUNK = 256
EPAD = 323584
NCHUNK = EPAD // NS // CHUNK
RPS = NPAD // NS
ZR = 64
BR = 128
COMP_OUT = KTOP + NC * NS * LN

_f32 = jnp.float32
_i32 = jnp.int32


def _sc_mesh():
    return plsc.VectorSubcoreMesh(
        core_axis_name="c", subcore_axis_name="s", num_cores=NC, num_subcores=NS)


def _make_segsum(with_cnt):
    d = 128
    out_type = [jax.ShapeDtypeStruct((NC, NPAD, d), _f32)]
    if with_cnt:
        out_type.append(jax.ShapeDtypeStruct((NC, NPAD), _f32))
    scratch = [
        pltpu.VMEM((CHUNK,), _i32),
        pltpu.VMEM((CHUNK,), _i32),
        pltpu.VMEM((CHUNK, d), _f32),
        pltpu.VMEM((ZR, d), _f32),
        pltpu.VMEM_SHARED((NPAD, d), _f32),
    ]
    if with_cnt:
        scratch += [
            pltpu.VMEM((CHUNK,), _f32),
            pltpu.VMEM((RPS,), _f32),
            pltpu.VMEM_SHARED((NPAD,), _f32),
        ]
    scratch.append(pltpu.SemaphoreType.DMA)

    @functools.partial(pl.kernel, out_type=tuple(out_type), mesh=_sc_mesh(),
                       scratch_types=scratch)
    def k(tab0, tab1, src0, dst0, src1, dst1, *rest):
        if with_cnt:
            (agg_out, cnt_out, sidx, didx, rows, zbuf, acc,
             ones, cstage, cacc, sem) = rest
        else:
            agg_out, sidx, didx, rows, zbuf, acc, sem = rest
        c = lax.axis_index("c")
        s = lax.axis_index("s")

        nz = (ZR * d) // LN
        def zstep(i, _):
            r = i // (d // LN)
            col = (i % (d // LN)) * LN
            zbuf[r, pl.ds(col, LN)] = jnp.zeros((LN,), _f32)
            return 0
        lax.fori_loop(0, nz, zstep, 0)
        base_r = s * RPS
        def zcp(j, _):
            pltpu.sync_copy(zbuf, acc.at[pl.ds(base_r + j * ZR, ZR)])
            return 0
        lax.fori_loop(0, RPS // ZR, zcp, 0)
        if with_cnt:
            def ofill(i, _):
                ones[pl.ds(i * LN, LN)] = jnp.ones((LN,), _f32)
                return 0
            lax.fori_loop(0, CHUNK // LN, ofill, 0)
            def cfill(i, _):
                cstage[pl.ds(i * LN, LN)] = jnp.zeros((LN,), _f32)
                return 0
            lax.fori_loop(0, RPS // LN, cfill, 0)
            pltpu.sync_copy(cstage, cacc.at[pl.ds(base_r, RPS)])
        plsc.subcore_barrier()

        ebase = s * (EPAD // NS)
        def do_edges(tab, srcr, dstr):
            def step(kk, _):
                off = ebase + kk * CHUNK
                pltpu.sync_copy(srcr.at[pl.ds(off, CHUNK)], sidx)
                pltpu.sync_copy(dstr.at[pl.ds(off, CHUNK)], didx)
                pltpu.async_copy(tab.at[sidx], rows, sem).wait()
                pltpu.sync_copy(rows, acc.at[didx], add=True)
                if with_cnt:
                    pltpu.sync_copy(ones, cacc.at[didx], add=True)
                return 0
            lax.fori_loop(0, NCHUNK, step, 0)

        @pl.when(c == 0)
        def _():
            do_edges(tab0, src0, dst0)

        @pl.when(c == 1)
        def _():
            do_edges(tab1, src1, dst1)

        plsc.subcore_barrier()

        pltpu.sync_copy(acc.at[pl.ds(base_r, RPS)],
                        agg_out.at[c, pl.ds(base_r, RPS)])
        if with_cnt:
            pltpu.sync_copy(cacc.at[pl.ds(base_r, RPS)],
                            cnt_out.at[c, pl.ds(base_r, RPS)])

    return k


def _layer1_body(agg, cnt, xa1, xa2, wl, wr, b, h1a, h1b):
    c0 = cnt[0]
    c1 = cnt[1]
    inv0 = 1.0 / jnp.maximum(c0, 1.0)
    inv1 = 1.0 / jnp.maximum(c1, 1.0)
    wlv = wl[...]
    wrv = wr[...]
    bv = b[...]
    h1a[...] = jnp.maximum(
        jnp.dot(agg[0] * inv0, wlv, preferred_element_type=_f32)
        + jnp.dot(xa1[...], wrv, preferred_element_type=_f32) + bv, 0.0)
    h1b[...] = jnp.maximum(
        jnp.dot(agg[1] * inv1, wlv, preferred_element_type=_f32)
        + jnp.dot(xa2[...], wrv, preferred_element_type=_f32) + bv, 0.0)


def _layer2_body(agg, cnt, h1a, h1b, wl, wr, b, sim):
    i = pl.program_id(0)
    c0 = cnt[0]
    c1 = cnt[1]
    inv0 = 1.0 / jnp.maximum(c0, 1.0)
    inv1 = 1.0 / jnp.maximum(c1, 1.0)
    wlv = wl[...]
    wrv = wr[...]
    bv = b[...]
    o1 = jnp.maximum(
        jnp.dot(agg[0] * inv0, wlv, preferred_element_type=_f32)
        + jnp.dot(h1a[...], wrv, preferred_element_type=_f32) + bv, 0.0)
    o2 = jnp.maximum(
        jnp.dot(agg[1] * inv1, wlv, preferred_element_type=_f32)
        + jnp.dot(h1b[...], wrv, preferred_element_type=_f32) + bv, 0.0)
    d = o1 - o2 + 1e-6
    s2 = jnp.sum(d * d, axis=1, keepdims=True)
    row = i * BR + lax.broadcasted_iota(_i32, (BR, 1), 0)
    sim[...] = jnp.where(row < NV, jnp.sqrt(s2), -1.0)


def _bisect_body(simr, t_out, off_out):
    v = simr[...]
    bu = lax.bitcast_convert_type(v, jnp.uint32)
    neg = (bu >> jnp.uint32(31)) == jnp.uint32(1)
    ku = bu ^ jnp.where(neg, jnp.uint32(0xFFFFFFFF), jnp.uint32(0x80000000))

    def step(i, t):
        cand = t | (jnp.uint32(1) << (jnp.uint32(31) - i.astype(jnp.uint32)))
        cnt = jnp.sum((ku >= cand).astype(_i32))
        return jnp.where(cnt >= KTOP, cand, t)

    t = lax.fori_loop(0, 32, step, jnp.uint32(0))
    tb = jnp.where((t >> jnp.uint32(31)) == jnp.uint32(1),
                   t ^ jnp.uint32(0x80000000), t ^ jnp.uint32(0xFFFFFFFF))
    tf = lax.bitcast_convert_type(tb, _f32)
    t_out[...] = jnp.broadcast_to(tf, (1, 1))

    c_sl = jnp.sum(jnp.where(v > tf, 1.0, 0.0), axis=1)
    ri = lax.broadcasted_iota(_i32, (NS, NS), 0)
    tril = jnp.where(ri > lax.broadcasted_iota(_i32, (NS, NS), 1), 1.0, 0.0)
    colsum = jnp.sum(c_sl, axis=0, keepdims=True)
    acol = jnp.dot(colsum, tril.T, preferred_element_type=_f32)
    brow = jnp.dot(tril, c_sl, preferred_element_type=_f32)
    off_out[...] = (acol + brow).astype(_i32)


def _make_compact():
    vps = NPAD // NS
    nvr = vps // LN
    fill = COMP_OUT // NS

    scratch = [
        pltpu.VMEM((vps,), _f32),
        pltpu.VMEM((LN,), _f32),
        pltpu.VMEM((vps,), _f32),
        pltpu.VMEM((vps,), _i32),
        pltpu.VMEM((fill,), _f32),
        pltpu.VMEM((LN,), _i32),
        pltpu.VMEM_SHARED((COMP_OUT,), _f32),
        pltpu.SemaphoreType.DMA,
    ]

    @functools.partial(
        pl.kernel,
        out_type=jax.ShapeDtypeStruct((COMP_OUT,), _f32),
        mesh=_sc_mesh(), scratch_types=scratch)
    def k(sim_hbm, t_hbm, off_hbm, cand_out, vbuf, tbuf, svals, sidx, fbuf,
          offb, osp, sem):
        c = lax.axis_index("c")
        s = lax.axis_index("s")
        zeros = jnp.zeros((LN,), _i32)
        ones = jnp.ones((LN,), _i32)

        @pl.when(c == 0)
        def _():
            pltpu.sync_copy(t_hbm, tbuf)
            tvec = tbuf[...]
            def fstep(i, _):
                fbuf[pl.ds(i * LN, LN)] = tvec
                return 0
            lax.fori_loop(0, fill // LN, fstep, 0)
            pltpu.sync_copy(fbuf, osp.at[pl.ds(s * fill, fill)])
            pltpu.sync_copy(sim_hbm.at[pl.ds(s * vps, vps)], vbuf)
            pltpu.sync_copy(off_hbm.at[s], offb)
            plsc.subcore_barrier()

            dump = KTOP + s * LN + lax.iota(_i32, LN)
            def sstep(i, off):
                v = vbuf[pl.ds(i * LN, LN)]
                m = v > tvec
                idx = jnp.where(m, off, dump)
                svals[pl.ds(i * LN, LN)] = v
                sidx[pl.ds(i * LN, LN)] = idx
                return off + jnp.where(m, ones, zeros)
            lax.fori_loop(0, nvr, sstep, offb[...])
            pltpu.sync_copy(svals, osp.at[sidx])
            plsc.subcore_barrier()
            pltpu.sync_copy(osp.at[pl.ds(s * fill, fill)], fbuf)
            pltpu.sync_copy(fbuf, cand_out.at[pl.ds(s * fill, fill)])

    return k


def _rank_body(cv_ref, cs_ref, r_ref):
    cv = cv_ref[...]
    ids = (lax.broadcasted_iota(_i32, (8, 128), 0) * 128
           + lax.broadcasted_iota(_i32, (8, 128), 1))

    def step(j, acc):
        tj = cs_ref[j]
        gt = (tj > cv).astype(_i32)
        eq = (jnp.logical_and(tj == cv, ids > j)).astype(_i32)
        return acc + gt + eq

    r_ref[...] = lax.fori_loop(0, KTOP, step, jnp.zeros((8, 128), _i32))


def _head_body(cs_ref, rs_ref, w1, b1, w2, b2, w3, b3, out_ref):
    pids = (lax.broadcasted_iota(_i32, (8, 128), 0) * 128
            + lax.broadcasted_iota(_i32, (8, 128), 1))

    def step(i, sv):
        return jnp.where(pids == rs_ref[i], cs_ref[i], sv)

    sorted_v = lax.fori_loop(0, KTOP, step, jnp.zeros((8, 128), _f32))
    h = jnp.zeros((1, 128), _f32)
    for si in range(8):
        h = h + jnp.dot(sorted_v[si:si + 1, :], w1[si],
                        preferred_element_type=_f32)
    h = jnp.maximum(h + b1[...], 0.0)
    h = jnp.maximum(jnp.dot(h, w2[...], preferred_element_type=_f32) + b2[...], 0.0)
    o = jnp.dot(h, w3[...], preferred_element_type=_f32) + b3[...]
    out_ref[...] = 1.0 / (1.0 + jnp.exp(-o))


_segsum_l1 = _make_segsum(True)
_segsum_l2 = _make_segsum(False)
_compact = _make_compact()

_layer1 = pl.pallas_call(
    _layer1_body,
    grid=(NPAD // BR,),
    in_specs=[
        pl.BlockSpec((NC, BR, 128), lambda i: (0, i, 0)),
        pl.BlockSpec((NC, BR, 1), lambda i: (0, i, 0)),
        pl.BlockSpec((BR, 128), lambda i: (i, 0)),
        pl.BlockSpec((BR, 128), lambda i: (i, 0)),
        pl.BlockSpec((128, 128), lambda i: (0, 0)),
        pl.BlockSpec((128, 128), lambda i: (0, 0)),
        pl.BlockSpec((1, 128), lambda i: (0, 0)),
    ],
    out_specs=[
        pl.BlockSpec((BR, 128), lambda i: (i, 0)),
        pl.BlockSpec((BR, 128), lambda i: (i, 0)),
    ],
    out_shape=[
        jax.ShapeDtypeStruct((NPAD, 128), _f32),
        jax.ShapeDtypeStruct((NPAD, 128), _f32),
    ],
)

_layer2 = pl.pallas_call(
    _layer2_body,
    grid=(NPAD // BR,),
    in_specs=[
        pl.BlockSpec((NC, BR, 128), lambda i: (0, i, 0)),
        pl.BlockSpec((NC, BR, 1), lambda i: (0, i, 0)),
        pl.BlockSpec((BR, 128), lambda i: (i, 0)),
        pl.BlockSpec((BR, 128), lambda i: (i, 0)),
        pl.BlockSpec((128, H2), lambda i: (0, 0)),
        pl.BlockSpec((128, H2), lambda i: (0, 0)),
        pl.BlockSpec((1, H2), lambda i: (0, 0)),
    ],
    out_specs=pl.BlockSpec((BR, 1), lambda i: (i, 0)),
    out_shape=jax.ShapeDtypeStruct((NPAD, 1), _f32),
)

_bisect = pl.pallas_call(
    _bisect_body,
    out_shape=[
        jax.ShapeDtypeStruct((1, 1), _f32),
        jax.ShapeDtypeStruct((NS, LN), _i32),
    ],
)

_rank = pl.pallas_call(
    _rank_body,
    in_specs=[
        pl.BlockSpec(memory_space=pltpu.VMEM),
        pl.BlockSpec(memory_space=pltpu.SMEM),
    ],
    out_shape=jax.ShapeDtypeStruct((8, 128), _i32),
)

_head = pl.pallas_call(
    _head_body,
    in_specs=[
        pl.BlockSpec(memory_space=pltpu.SMEM),
        pl.BlockSpec(memory_space=pltpu.SMEM),
    ] + [pl.BlockSpec(memory_space=pltpu.VMEM)] * 6,
    out_shape=jax.ShapeDtypeStruct((1, 1), _f32),
)


def _pad_edges(ei):
    npe = EPAD - NE
    ar = jnp.arange(npe, dtype=_i32)
    src = jnp.concatenate([ei[0], ar % NV])
    dst = jnp.concatenate([ei[1], NV + (ar % (NPAD - NV))])
    return src, dst


def kernel(x1, x2, edge_index1, edge_index2, params):
    p = params
    x1p = jnp.pad(x1, ((0, NPAD - NV), (0, 0)))
    x2p = jnp.pad(x2, ((0, NPAD - NV), (0, 0)))
    s1, d1 = _pad_edges(edge_index1)
    s2, d2 = _pad_edges(edge_index2)

    wl1 = p['Wl1'].T
    wr1 = p['Wr1'].T
    b1 = p['b1'].reshape(1, 128)
    wl2 = p['Wl2'].T
    wr2 = p['Wr2'].T
    b2 = p['b2'].reshape(1, H2)
    w1eff = (p['W_fc1'].T * p['g1'][None, :]).reshape(8, 128, 128)
    b1eff = (p['b_fc1'] * p['g1'] + p['be1']).reshape(1, 128)
    w2eff = p['W_fc2'].T * p['g2'][None, :]
    b2eff = (p['b_fc2'] * p['g2'] + p['be2']).reshape(1, H2)
    w3 = p['W_fc3'].T
    b3 = p['b_fc3'].reshape(1, 1)

    agg1, cnt2 = _segsum_l1(x1, x2, s1, d1, s2, d2)
    cnt = cnt2.reshape(NC, NPAD, 1)
    h1a, h1b = _layer1(agg1, cnt, x1p, x2p, wl1, wr1, b1)
    agg2, = _segsum_l2(h1a, h1b, s1, d1, s2, d2)
    sim = _layer2(agg2, cnt, h1a, h1b, wl2, wr2, b2)

    t, off0 = _bisect(sim.reshape(NS, NPAD // NS // LN, LN))
    cand = _compact(sim.reshape(NPAD), jnp.broadcast_to(t.reshape(()), (LN,)),
                    off0)
    c1024 = cand[:KTOP]
    r = _rank(c1024.reshape(8, 128), c1024)
    out = _head(c1024, r.reshape(KTOP), w1eff, b1eff, w2eff, b2eff, w3, b3)
    return out.reshape(1)


# speedup vs baseline: 7.9818x; 1.5272x over previous
import functools

import jax
import jax.numpy as jnp
from jax import lax
from jax.experimental import pallas as pl
from jax.experimental.pallas import tpu as pltpu
from jax.experimental.pallas import tpu_sc as plsc

NV = 10000
NE = 320000
DIN = 128
H1 = 128
H2 = 256
KTOP = 1024

NC, NS, LN = 2, 16, 16
NPAD = 10240
DAUG = 144
CHUNK = 128
EPAD = 327680
NCHUNK = EPAD // NS // CHUNK
RPS = NPAD // NS
ZR = 32
BR = 128
COMP_OUT = KTOP + NC * NS * LN

_f32 = jnp.float32
_i32 = jnp.int32


def _sc_mesh():
    return plsc.VectorSubcoreMesh(
        core_axis_name="c", subcore_axis_name="s", num_cores=NC, num_subcores=NS)


def _make_segsum(with_cnt):
    d = 128
    out_type = [jax.ShapeDtypeStruct((NC, NPAD, d), _f32)]
    if with_cnt:
        out_type.append(jax.ShapeDtypeStruct((NC, NPAD), _f32))
    scratch = [
        [pltpu.VMEM((CHUNK,), _i32)] * 4,
        [pltpu.VMEM((CHUNK,), _i32)] * 4,
        pltpu.VMEM((CHUNK, d), _f32),
        pltpu.VMEM((CHUNK, d), _f32),
        pltpu.VMEM((ZR, d), _f32),
        pltpu.VMEM_SHARED((NPAD, d), _f32),
    ]
    if with_cnt:
        scratch += [
            pltpu.VMEM((CHUNK,), _f32),
            pltpu.VMEM((RPS,), _f32),
            pltpu.VMEM_SHARED((NPAD,), _f32),
        ]
    scratch += [[pltpu.SemaphoreType.DMA] * 4] * 2
    scratch += [pltpu.SemaphoreType.DMA] * 6

    @functools.partial(pl.kernel, out_type=tuple(out_type), mesh=_sc_mesh(),
                       scratch_types=scratch)
    def k(tab0, tab1, src0, dst0, src1, dst1, *rest):
        if with_cnt:
            (agg_out, cnt_out, sslot, dslot, rowsA, rowsB, zbuf, acc,
             ones, cstage, cacc, isem, idsem,
             gsA, gsB, ssA, ssB, csA, csB) = rest
        else:
            (agg_out, sslot, dslot, rowsA, rowsB, zbuf, acc, isem, idsem,
             gsA, gsB, ssA, ssB, csA, csB) = rest
        c = lax.axis_index("c")
        s = lax.axis_index("s")

        nz = (ZR * d) // LN
        def zstep(i, _):
            r = i // (d // LN)
            col = (i % (d // LN)) * LN
            zbuf[r, pl.ds(col, LN)] = jnp.zeros((LN,), _f32)
            return 0
        lax.fori_loop(0, nz, zstep, 0)
        base_r = s * RPS
        def zcp(j, _):
            pltpu.sync_copy(zbuf, acc.at[pl.ds(base_r + j * ZR, ZR)])
            return 0
        lax.fori_loop(0, RPS // ZR, zcp, 0)
        if with_cnt:
            def ofill(i, _):
                ones[pl.ds(i * LN, LN)] = jnp.ones((LN,), _f32)
                return 0
            lax.fori_loop(0, CHUNK // LN, ofill, 0)
            def cfill(i, _):
                cstage[pl.ds(i * LN, LN)] = jnp.zeros((LN,), _f32)
                return 0
            lax.fori_loop(0, RPS // LN, cfill, 0)
            pltpu.sync_copy(cstage, cacc.at[pl.ds(base_r, RPS)])
        plsc.subcore_barrier()

        def do_edges(tab, srcr, dstr):
            def load_idx(j, m):
                pltpu.async_copy(srcr.at[s, j], sslot[m], isem[m])
                pltpu.async_copy(dstr.at[s, j], dslot[m], idsem[m])

            def wait_idx(m):
                pltpu.make_async_copy(srcr.at[s, 0], sslot[m],
                                      isem[m]).wait()
                pltpu.make_async_copy(dstr.at[s, 0], dslot[m],
                                      idsem[m]).wait()

            def gather(m, rows, sem):
                pltpu.async_copy(tab.at[sslot[m]], rows, sem)

            def scat(m, rows, sem, semc):
                pltpu.async_copy(rows, acc.at[dslot[m]], sem, add=True)
                if with_cnt:
                    pltpu.async_copy(ones, cacc.at[dslot[m]], semc,
                                     add=True)

            def wait_g(rows, sem):
                pltpu.make_async_copy(tab.at[pl.ds(0, CHUNK)], rows,
                                      sem).wait()

            def wait_s(rows, sem, semc):
                pltpu.make_async_copy(tab.at[pl.ds(0, CHUNK)], rows,
                                      sem).wait()
                if with_cnt:
                    pltpu.make_async_copy(cnt_out.at[0, pl.ds(0, CHUNK)],
                                          ones, semc).wait()

            rows = (rowsA, rowsB)
            gs = (gsA, gsB)
            ss = (ssA, ssB)
            cs = (csA, csB)

            def chunk(j, jp, do1, do2, do3, do4):
                p = jp % 2
                m = jp % 4
                if do1:
                    wait_s(rows[p ^ 1], ss[p ^ 1], cs[p ^ 1])
                if do2:
                    wait_idx((m + 1) % 4)
                if do3:
                    load_idx(j + 3, (m + 3) % 4)
                if do4:
                    gather((m + 1) % 4, rows[p ^ 1], gs[p ^ 1])
                wait_g(rows[p], gs[p])
                scat(m, rows[p], ss[p], cs[p])

            for m0 in range(3):
                load_idx(m0, m0)
            for m0 in range(3):
                wait_idx(m0)
            gather(0, rowsA, gsA)
            chunk(0, 0, False, False, True, True)
            chunk(1, 1, True, False, True, True)
            chunk(2, 2, True, True, True, True)
            chunk(3, 3, True, True, True, True)

            def body(kb, _):
                j = 4 * kb
                chunk(j + 0, 0, True, True, True, True)
                chunk(j + 1, 1, True, True, True, True)
                chunk(j + 2, 2, True, True, True, True)
                chunk(j + 3, 3, True, True, True, True)
                return 0
            lax.fori_loop(1, NCHUNK // 4 - 1, body, 0)

            j = NCHUNK - 4
            chunk(j + 0, 0, True, True, True, True)
            chunk(j + 1, 1, True, True, False, True)
            chunk(j + 2, 2, True, True, False, True)
            chunk(j + 3, 3, True, False, False, False)
            wait_s(rowsB, ssB, csB)

        @pl.when(c == 0)
        def _():
            do_edges(tab0, src0, dst0)

        @pl.when(c == 1)
        def _():
            do_edges(tab1, src1, dst1)

        plsc.subcore_barrier()

        pltpu.sync_copy(acc.at[pl.ds(base_r, RPS)],
                        agg_out.at[c, pl.ds(base_r, RPS)])
        if with_cnt:
            pltpu.sync_copy(cacc.at[pl.ds(base_r, RPS)],
                            cnt_out.at[c, pl.ds(base_r, RPS)])

    return k


def _layer1_body(agg, cnt, xa1, xa2, wl, wr, b, h1a, h1b):
    c0 = cnt[0]
    c1 = cnt[1]
    inv0 = 1.0 / jnp.maximum(c0, 1.0)
    inv1 = 1.0 / jnp.maximum(c1, 1.0)
    wlv = wl[...]
    wrv = wr[...]
    bv = b[...]
    h1a[...] = jnp.maximum(
        jnp.dot(agg[0] * inv0, wlv, preferred_element_type=_f32)
        + jnp.dot(xa1[...], wrv, preferred_element_type=_f32) + bv, 0.0)
    h1b[...] = jnp.maximum(
        jnp.dot(agg[1] * inv1, wlv, preferred_element_type=_f32)
        + jnp.dot(xa2[...], wrv, preferred_element_type=_f32) + bv, 0.0)


def _layer2_body(agg, cnt, h1a, h1b, wl, wr, b, sim):
    i = pl.program_id(0)
    c0 = cnt[0]
    c1 = cnt[1]
    inv0 = 1.0 / jnp.maximum(c0, 1.0)
    inv1 = 1.0 / jnp.maximum(c1, 1.0)
    wlv = wl[...]
    wrv = wr[...]
    bv = b[...]
    o1 = jnp.maximum(
        jnp.dot(agg[0] * inv0, wlv, preferred_element_type=_f32)
        + jnp.dot(h1a[...], wrv, preferred_element_type=_f32) + bv, 0.0)
    o2 = jnp.maximum(
        jnp.dot(agg[1] * inv1, wlv, preferred_element_type=_f32)
        + jnp.dot(h1b[...], wrv, preferred_element_type=_f32) + bv, 0.0)
    d = o1 - o2 + 1e-6
    s2 = jnp.sum(d * d, axis=1, keepdims=True)
    row = i * BR + lax.broadcasted_iota(_i32, (BR, 1), 0)
    sim[...] = jnp.where(row < NV, jnp.sqrt(s2), -1.0)


def _bisect_body(simr, t_out, off_out):
    v = simr[...]
    bu = lax.bitcast_convert_type(v, jnp.uint32)
    neg = (bu >> jnp.uint32(31)) == jnp.uint32(1)
    ku = bu ^ jnp.where(neg, jnp.uint32(0xFFFFFFFF), jnp.uint32(0x80000000))

    def step(i, t):
        cand = t | (jnp.uint32(1) << (jnp.uint32(31) - i.astype(jnp.uint32)))
        cnt = jnp.sum((ku >= cand).astype(_i32))
        return jnp.where(cnt >= KTOP, cand, t)

    t = lax.fori_loop(0, 32, step, jnp.uint32(0))
    tb = jnp.where((t >> jnp.uint32(31)) == jnp.uint32(1),
                   t ^ jnp.uint32(0x80000000), t ^ jnp.uint32(0xFFFFFFFF))
    tf = lax.bitcast_convert_type(tb, _f32)
    t_out[...] = jnp.broadcast_to(tf, (1, 1))

    c_sl = jnp.sum(jnp.where(v > tf, 1.0, 0.0), axis=1)
    ri = lax.broadcasted_iota(_i32, (NS, NS), 0)
    tril = jnp.where(ri > lax.broadcasted_iota(_i32, (NS, NS), 1), 1.0, 0.0)
    colsum = jnp.sum(c_sl, axis=0, keepdims=True)
    acol = jnp.dot(colsum, tril.T, preferred_element_type=_f32)
    brow = jnp.dot(tril, c_sl, preferred_element_type=_f32)
    off_out[...] = (acol + brow).astype(_i32)


def _make_compact():
    vps = NPAD // NS
    nvr = vps // LN
    fill = COMP_OUT // NS

    scratch = [
        pltpu.VMEM((vps,), _f32),
        pltpu.VMEM((LN,), _f32),
        pltpu.VMEM((vps,), _f32),
        pltpu.VMEM((vps,), _i32),
        pltpu.VMEM((fill,), _f32),
        pltpu.VMEM((LN,), _i32),
        pltpu.VMEM_SHARED((COMP_OUT,), _f32),
        pltpu.SemaphoreType.DMA,
    ]

    @functools.partial(
        pl.kernel,
        out_type=jax.ShapeDtypeStruct((COMP_OUT,), _f32),
        mesh=_sc_mesh(), scratch_types=scratch)
    def k(sim_hbm, t_hbm, off_hbm, cand_out, vbuf, tbuf, svals, sidx, fbuf,
          offb, osp, sem):
        c = lax.axis_index("c")
        s = lax.axis_index("s")
        zeros = jnp.zeros((LN,), _i32)
        ones = jnp.ones((LN,), _i32)

        @pl.when(c == 0)
        def _():
            pltpu.sync_copy(t_hbm, tbuf)
            tvec = tbuf[...]
            def fstep(i, _):
                fbuf[pl.ds(i * LN, LN)] = tvec
                return 0
            lax.fori_loop(0, fill // LN, fstep, 0)
            pltpu.sync_copy(fbuf, osp.at[pl.ds(s * fill, fill)])
            pltpu.sync_copy(sim_hbm.at[pl.ds(s * vps, vps)], vbuf)
            pltpu.sync_copy(off_hbm.at[s], offb)
            plsc.subcore_barrier()

            dump = KTOP + s * LN + lax.iota(_i32, LN)
            def sstep(i, off):
                v = vbuf[pl.ds(i * LN, LN)]
                m = v > tvec
                idx = jnp.where(m, off, dump)
                svals[pl.ds(i * LN, LN)] = v
                sidx[pl.ds(i * LN, LN)] = idx
                return off + jnp.where(m, ones, zeros)
            lax.fori_loop(0, nvr, sstep, offb[...])
            pltpu.sync_copy(svals, osp.at[sidx])
            plsc.subcore_barrier()
            pltpu.sync_copy(osp.at[pl.ds(s * fill, fill)], fbuf)
            pltpu.sync_copy(fbuf, cand_out.at[pl.ds(s * fill, fill)])

    return k


def _rank_body(cv_ref, cs_ref, r_ref):
    cv = cv_ref[...]
    ids = (lax.broadcasted_iota(_i32, (8, 128), 0) * 128
           + lax.broadcasted_iota(_i32, (8, 128), 1))

    def step(j, acc):
        tj = cs_ref[j]
        gt = (tj > cv).astype(_i32)
        eq = (jnp.logical_and(tj == cv, ids > j)).astype(_i32)
        return acc + gt + eq

    r_ref[...] = lax.fori_loop(0, KTOP, step, jnp.zeros((8, 128), _i32))


def _head_body(cs_ref, rs_ref, w1, b1, w2, b2, w3, b3, out_ref):
    pids = (lax.broadcasted_iota(_i32, (8, 128), 0) * 128
            + lax.broadcasted_iota(_i32, (8, 128), 1))

    def step(i, sv):
        return jnp.where(pids == rs_ref[i], cs_ref[i], sv)

    sorted_v = lax.fori_loop(0, KTOP, step, jnp.zeros((8, 128), _f32))
    h = jnp.zeros((1, 128), _f32)
    for si in range(8):
        h = h + jnp.dot(sorted_v[si:si + 1, :], w1[si],
                        preferred_element_type=_f32)
    h = jnp.maximum(h + b1[...], 0.0)
    h = jnp.maximum(jnp.dot(h, w2[...], preferred_element_type=_f32) + b2[...], 0.0)
    o = jnp.dot(h, w3[...], preferred_element_type=_f32) + b3[...]
    out_ref[...] = 1.0 / (1.0 + jnp.exp(-o))


_segsum_l1 = _make_segsum(True)
_segsum_l2 = _make_segsum(False)
_compact = _make_compact()

_layer1 = pl.pallas_call(
    _layer1_body,
    grid=(NPAD // BR,),
    in_specs=[
        pl.BlockSpec((NC, BR, 128), lambda i: (0, i, 0)),
        pl.BlockSpec((NC, BR, 1), lambda i: (0, i, 0)),
        pl.BlockSpec((BR, 128), lambda i: (i, 0)),
        pl.BlockSpec((BR, 128), lambda i: (i, 0)),
        pl.BlockSpec((128, 128), lambda i: (0, 0)),
        pl.BlockSpec((128, 128), lambda i: (0, 0)),
        pl.BlockSpec((1, 128), lambda i: (0, 0)),
    ],
    out_specs=[
        pl.BlockSpec((BR, 128), lambda i: (i, 0)),
        pl.BlockSpec((BR, 128), lambda i: (i, 0)),
    ],
    out_shape=[
        jax.ShapeDtypeStruct((NPAD, 128), _f32),
        jax.ShapeDtypeStruct((NPAD, 128), _f32),
    ],
)

_layer2 = pl.pallas_call(
    _layer2_body,
    grid=(NPAD // BR,),
    in_specs=[
        pl.BlockSpec((NC, BR, 128), lambda i: (0, i, 0)),
        pl.BlockSpec((NC, BR, 1), lambda i: (0, i, 0)),
        pl.BlockSpec((BR, 128), lambda i: (i, 0)),
        pl.BlockSpec((BR, 128), lambda i: (i, 0)),
        pl.BlockSpec((128, H2), lambda i: (0, 0)),
        pl.BlockSpec((128, H2), lambda i: (0, 0)),
        pl.BlockSpec((1, H2), lambda i: (0, 0)),
    ],
    out_specs=pl.BlockSpec((BR, 1), lambda i: (i, 0)),
    out_shape=jax.ShapeDtypeStruct((NPAD, 1), _f32),
)

_bisect = pl.pallas_call(
    _bisect_body,
    out_shape=[
        jax.ShapeDtypeStruct((1, 1), _f32),
        jax.ShapeDtypeStruct((NS, LN), _i32),
    ],
)

_rank = pl.pallas_call(
    _rank_body,
    in_specs=[
        pl.BlockSpec(memory_space=pltpu.VMEM),
        pl.BlockSpec(memory_space=pltpu.SMEM),
    ],
    out_shape=jax.ShapeDtypeStruct((8, 128), _i32),
)

_head = pl.pallas_call(
    _head_body,
    in_specs=[
        pl.BlockSpec(memory_space=pltpu.SMEM),
        pl.BlockSpec(memory_space=pltpu.SMEM),
    ] + [pl.BlockSpec(memory_space=pltpu.VMEM)] * 6,
    out_shape=jax.ShapeDtypeStruct((1, 1), _f32),
)


def _pad_edges(ei):
    npe = EPAD - NE
    ar = jnp.arange(npe, dtype=_i32)
    src = jnp.concatenate([ei[0], ar % NV]).reshape(NS, NCHUNK, CHUNK)
    dst = jnp.concatenate([ei[1], NV + (ar % (NPAD - NV))]
                          ).reshape(NS, NCHUNK, CHUNK)
    return src, dst


def kernel(x1, x2, edge_index1, edge_index2, params):
    p = params
    x1p = jnp.pad(x1, ((0, NPAD - NV), (0, 0)))
    x2p = jnp.pad(x2, ((0, NPAD - NV), (0, 0)))
    s1, d1 = _pad_edges(edge_index1)
    s2, d2 = _pad_edges(edge_index2)

    wl1 = p['Wl1'].T
    wr1 = p['Wr1'].T
    b1 = p['b1'].reshape(1, 128)
    wl2 = p['Wl2'].T
    wr2 = p['Wr2'].T
    b2 = p['b2'].reshape(1, H2)
    w1eff = (p['W_fc1'].T * p['g1'][None, :]).reshape(8, 128, 128)
    b1eff = (p['b_fc1'] * p['g1'] + p['be1']).reshape(1, 128)
    w2eff = p['W_fc2'].T * p['g2'][None, :]
    b2eff = (p['b_fc2'] * p['g2'] + p['be2']).reshape(1, H2)
    w3 = p['W_fc3'].T
    b3 = p['b_fc3'].reshape(1, 1)

    agg1, cnt2 = _segsum_l1(x1, x2, s1, d1, s2, d2)
    cnt = cnt2.reshape(NC, NPAD, 1)
    h1a, h1b = _layer1(agg1, cnt, x1p, x2p, wl1, wr1, b1)
    agg2, = _segsum_l2(h1a, h1b, s1, d1, s2, d2)
    sim = _layer2(agg2, cnt, h1a, h1b, wl2, wr2, b2)

    t, off0 = _bisect(sim.reshape(NS, NPAD // NS // LN, LN))
    cand = _compact(sim.reshape(NPAD), jnp.broadcast_to(t.reshape(()), (LN,)),
                    off0)
    c1024 = cand[:KTOP]
    r = _rank(c1024.reshape(8, 128), c1024)
    out = _head(c1024, r.reshape(KTOP), w1eff, b1eff, w2eff, b2eff, w3, b3)
    return out.reshape(1)
